# baseline (speedup 1.0000x reference)
import numpy as np

import jax
import jax.numpy as jnp
from jax.experimental import pallas as pl
from jax.experimental.pallas import tpu as pltpu

_HO = 13
_F = 256
_MO = 208
_MY = 224
_PB = 32

_TAP = {0: (0, 0), 1: (1, 0), 2: (0, 1)}


def _mask_np():
    m = np.zeros((2, 2, _F, 192), np.float32)
    for u in range(2):
        for v in range(2):
            pm = 14 if u == 0 else 13
            qm = 14 if v == 0 else 13
            m2 = np.zeros((16, 16), np.float32)
            m2[:pm, :qm] = 1.0
            m[u, v] = np.broadcast_to(m2.reshape(_F, 1), (_F, 192))
    return m


def _sel_np():
    s = np.zeros((_MO, _HO * _HO), np.float32)
    for n in range(_HO * _HO):
        s[16 * (n // _HO) + (n % _HO), n] = 1.0
    return s


_MASK = _mask_np()
_SEL = _sel_np()
_DN_T = (((0,), (0,)), ((), ()))


def _mega_kernel(x_ref, mask_ref, sel_ref, w0_ref, w1_ref, w2_ref, w3_ref,
                 b_ref, o_ref, p_ref, y2_ref, a0_ref):

    zhead = jnp.zeros((_PB, 192), jnp.bfloat16)
    b1 = b_ref[1, :, 0:192]
    for u in range(2):
        for v in range(2):
            y = jnp.dot(x_ref[0, u, v], w1_ref[...],
                        preferred_element_type=jnp.float32)
            y = jnp.maximum(y + b1, 0.0)
            p_ref[u, v, _PB:_PB + _F, :] = (
                y.astype(jnp.bfloat16) * mask_ref[u, v])
            p_ref[u, v, 0:_PB, :] = zhead
            p_ref[u, v, _PB + _F:, :] = zhead

    b2 = b_ref[2, :, 0:224]
    for u in range(2):
        for v in range(2):
            acc = jnp.zeros((_MY, 224), jnp.float32)
            for dh in range(3):
                e = u + dh - 1
                up, pa = e % 2, (e - (e % 2)) // 2
                for dw in range(3):
                    f = v + dw - 1
                    vp, qa = f % 2, (f - (f % 2)) // 2
                    ofs = _PB + 16 * pa + qa
                    acc = acc + jnp.dot(p_ref[up, vp, ofs:ofs + _MY, :],
                                        w2_ref[dh * 3 + dw],
                                        preferred_element_type=jnp.float32)
            y2 = jnp.maximum(acc + b2, 0.0)
            y2_ref[u, v, 0:_MY, :] = y2.astype(jnp.bfloat16)
            y2_ref[u, v, _MY:, :] = jnp.zeros((_F - _MY, 224), jnp.bfloat16)

    b3 = b_ref[3, :, 0:256]
    acc1 = jnp.zeros((_MO, 256), jnp.float32)
    for dh in range(3):
        up, pa = _TAP[dh]
        for dw in range(3):
            vp, qa = _TAP[dw]
            ofs = 16 * pa + qa
            acc1 = acc1 + jnp.dot(y2_ref[up, vp, ofs:ofs + _MO, :],
                                  w3_ref[dh * 3 + dw],
                                  preferred_element_type=jnp.float32)
    x1 = jnp.maximum(acc1 + b3, 0.0)

    b0 = b_ref[0, :, 0:384]
    acc0 = jnp.zeros((_MO, 384), jnp.float32)
    for dh in range(3):
        up, pa = _TAP[dh]
        for dw in range(3):
            vp, qa = _TAP[dw]
            ofs = 16 * pa + qa
            acc0 = acc0 + jnp.dot(x_ref[0, up, vp, ofs:ofs + _MO, :],
                                  w0_ref[dh * 3 + dw],
                                  preferred_element_type=jnp.float32)
    x0 = jnp.maximum(acc0 + b0, 0.0)

    for v in range(2):
        av = jnp.maximum(
            jnp.maximum(x_ref[0, 0, v, 0:_MY, :], x_ref[0, 1, v, 0:_MY, :]),
            x_ref[0, 0, v, 16:16 + _MY, :])
        if v == 0:
            a0_ref[...] = av
            a0v = av
        else:
            a1v = av
    mx = jnp.maximum(jnp.maximum(a0v[0:_MO, :], a1v[0:_MO, :]),
                     a0_ref[1:1 + _MO, :])

    sel = sel_ref[...]
    o_ref[0, 0:384, :] = jax.lax.dot_general(
        x0.astype(jnp.bfloat16), sel, _DN_T,
        preferred_element_type=jnp.float32)
    o_ref[0, 384:640, :] = jax.lax.dot_general(
        x1.astype(jnp.bfloat16), sel, _DN_T,
        preferred_element_type=jnp.float32)
    o_ref[0, 640:1024, :] = jax.lax.dot_general(
        mx, sel, _DN_T, preferred_element_type=jnp.float32)


def kernel(x, branch0_wk, branch0_b, branch1_0_wk, branch1_0_b,
           branch1_1_wk, branch1_1_b, branch1_2_wk, branch1_2_b):
    N = x.shape[0]
    xp = jnp.pad(x.astype(jnp.bfloat16), ((0, 0), (0, 0), (0, 5), (0, 5)))
    xp = xp.reshape(N, 384, 16, 2, 16, 2)
    xs2d = jnp.transpose(xp, (0, 3, 5, 2, 4, 1)).reshape(N, 2, 2, _F, 384)
    mask = jnp.asarray(_MASK, jnp.bfloat16)
    sel = jnp.asarray(_SEL, jnp.bfloat16)
    bias = jnp.stack([
        branch0_b,
        jnp.pad(branch1_0_b, (0, 192)),
        jnp.pad(branch1_1_b, (0, 160)),
        jnp.pad(branch1_2_b, (0, 128)),
    ]).reshape(4, 1, 384)

    out = pl.pallas_call(
        _mega_kernel,
        out_shape=jax.ShapeDtypeStruct((N, 1024, _HO * _HO), jnp.float32),
        grid_spec=pltpu.PrefetchScalarGridSpec(
            num_scalar_prefetch=0,
            grid=(N,),
            in_specs=[
                pl.BlockSpec((1, 2, 2, _F, 384), lambda n: (n, 0, 0, 0, 0)),
                pl.BlockSpec((2, 2, _F, 192), lambda n: (0, 0, 0, 0)),
                pl.BlockSpec((_MO, _HO * _HO), lambda n: (0, 0)),
                pl.BlockSpec((9, 384, 384), lambda n: (0, 0, 0)),
                pl.BlockSpec((384, 192), lambda n: (0, 0)),
                pl.BlockSpec((9, 192, 224), lambda n: (0, 0, 0)),
                pl.BlockSpec((9, 224, 256), lambda n: (0, 0, 0)),
                pl.BlockSpec((4, 1, 384), lambda n: (0, 0, 0)),
            ],
            out_specs=pl.BlockSpec((1, 1024, _HO * _HO), lambda n: (n, 0, 0)),
            scratch_shapes=[
                pltpu.VMEM((2, 2, 320, 192), jnp.bfloat16),
                pltpu.VMEM((2, 2, _F, 224), jnp.bfloat16),
                pltpu.VMEM((_MY, 384), jnp.bfloat16),
            ],
        ),
        compiler_params=pltpu.CompilerParams(
            dimension_semantics=("parallel",)),
    )(xs2d, mask, sel, branch0_wk, branch1_0_wk, branch1_1_wk,
      branch1_2_wk, bias)

    return out.reshape(N, 1024, _HO, _HO)
